# baseline (speedup 1.0000x reference)
import functools

import jax
import jax.numpy as jnp
from jax import lax
from jax.experimental import pallas as pl
from jax.experimental.pallas import tpu as pltpu
from jax.experimental.pallas import tpu_sc as plsc

_VOCAB = 1000
_HIDDEN = 128
_B = 4096
_SEQ = 20
_NTOK = _B * _SEQ

_NC = 2
_NS = 16
_NW = _NC * _NS

_PER_W = _NTOK // _NW
_CHUNK = 64
_NCHUNK = _PER_W // _CHUNK


def _table_body(we_ref, w_ref, b_ref, out_ref):
    out_ref[...] = lax.dot_general(
        we_ref[...], w_ref[...],
        (((1,), (1,)), ((), ())),
        preferred_element_type=jnp.float32,
    ) + b_ref[...]


def _build_table(we, W, b):
    return pl.pallas_call(
        _table_body,
        out_shape=jax.ShapeDtypeStruct((_VOCAB, _VOCAB), jnp.float32),
    )(we, W, b.reshape(1, _VOCAB))


def _gather_body(table_hbm, idx_hbm, out_hbm, idx_v, rows_v, gsem):
    wid = lax.axis_index("s") * _NC + lax.axis_index("c")
    base = wid * _PER_W
    pltpu.sync_copy(idx_hbm.at[wid], idx_v)

    @pl.loop(0, _NCHUNK)
    def _chunk(j):
        pltpu.async_copy(table_hbm.at[idx_v.at[j]], rows_v, gsem).wait()
        pltpu.sync_copy(rows_v, out_hbm.at[pl.ds(base + j * _CHUNK, _CHUNK)])


@functools.partial(
    pl.kernel,
    out_type=jax.ShapeDtypeStruct((_NTOK, _VOCAB), jnp.float32),
    mesh=plsc.VectorSubcoreMesh(core_axis_name="c", subcore_axis_name="s"),
    compiler_params=pltpu.CompilerParams(use_tc_tiling_on_sc=False),
    scratch_types=[
        pltpu.VMEM((_NCHUNK, _CHUNK), jnp.int32),
        pltpu.VMEM((_CHUNK, _VOCAB), jnp.float32),
        pltpu.SemaphoreType.DMA,
    ],
)
def _gather(table_hbm, idx_hbm, out_hbm, idx_v, rows_v, gsem):
    _gather_body(table_hbm, idx_hbm, out_hbm, idx_v, rows_v, gsem)


def kernel(x, we, W, b):
    table = _build_table(we, W, b)
    idx = x.astype(jnp.int32).reshape(_NW, _NCHUNK, _CHUNK)
    out = _gather(table, idx)
    return out.reshape(_B, _SEQ, _VOCAB)


# speedup vs baseline: 1.1705x; 1.1705x over previous
import functools

import jax
import jax.numpy as jnp
from jax import lax
from jax.experimental import pallas as pl
from jax.experimental.pallas import tpu as pltpu
from jax.experimental.pallas import tpu_sc as plsc

_VOCAB = 1000
_HIDDEN = 128
_B = 4096
_SEQ = 20
_NTOK = _B * _SEQ

_NC = 2
_NS = 16
_NW = _NC * _NS

_PER_W = _NTOK // _NW
_CHUNK = 32
_NCHUNK = _PER_W // _CHUNK


def _table_body(we_ref, w_ref, b_ref, out_ref):
    out_ref[...] = lax.dot_general(
        we_ref[...], w_ref[...],
        (((1,), (1,)), ((), ())),
        preferred_element_type=jnp.float32,
    ) + b_ref[...]


def _build_table(we, W, b):
    return pl.pallas_call(
        _table_body,
        out_shape=jax.ShapeDtypeStruct((_VOCAB, _VOCAB), jnp.float32),
    )(we, W, b.reshape(1, _VOCAB))


def _gather_body(table_hbm, idx_hbm, out_hbm, tab_s, idx_v, rows0, rows1,
                 gsem, ssem0, ssem1):
    cid = lax.axis_index("c")
    sid = lax.axis_index("s")
    wid = sid * _NC + cid
    base = wid * _PER_W
    rows = (rows0, rows1)
    ssem = (ssem0, ssem1)

    @pl.when(sid == 0)
    def _stage():
        pltpu.sync_copy(table_hbm, tab_s)

    pltpu.sync_copy(idx_hbm.at[wid], idx_v)
    plsc.subcore_barrier()

    def _do(j, p):
        pltpu.async_copy(tab_s.at[idx_v.at[j]], rows[p], gsem).wait()
        pltpu.async_copy(
            rows[p], out_hbm.at[pl.ds(base + j * _CHUNK, _CHUNK)], ssem[p])

    def _drain(p):
        pltpu.make_async_copy(
            rows[p], out_hbm.at[pl.ds(0, _CHUNK)], ssem[p]).wait()

    _do(0, 0)
    _do(1, 1)

    @pl.loop(2, _NCHUNK, step=2)
    def _chunks(g):
        for p in range(2):
            _drain(p)
            _do(g + p, p)

    _drain(0)
    _drain(1)


@functools.partial(
    pl.kernel,
    out_type=jax.ShapeDtypeStruct((_NTOK, _VOCAB), jnp.float32),
    mesh=plsc.VectorSubcoreMesh(core_axis_name="c", subcore_axis_name="s"),
    compiler_params=pltpu.CompilerParams(use_tc_tiling_on_sc=False),
    scratch_types=[
        pltpu.VMEM_SHARED((_VOCAB, _VOCAB), jnp.float32),
        pltpu.VMEM((_NCHUNK, _CHUNK), jnp.int32),
        pltpu.VMEM((_CHUNK, _VOCAB), jnp.float32),
        pltpu.VMEM((_CHUNK, _VOCAB), jnp.float32),
        pltpu.SemaphoreType.DMA,
        pltpu.SemaphoreType.DMA,
        pltpu.SemaphoreType.DMA,
    ],
)
def _gather(table_hbm, idx_hbm, out_hbm, tab_s, idx_v, rows0, rows1,
            gsem, ssem0, ssem1):
    _gather_body(table_hbm, idx_hbm, out_hbm, tab_s, idx_v, rows0, rows1,
                 gsem, ssem0, ssem1)


def kernel(x, we, W, b):
    table = _build_table(we, W, b)
    idx = x.astype(jnp.int32).reshape(_NW, _NCHUNK, _CHUNK)
    out = _gather(table, idx)
    return out.reshape(_B, _SEQ, _VOCAB)
